# baseline (speedup 1.0000x reference)
import jax
import jax.numpy as jnp
from jax.experimental import pallas as pl

_BLOCK = 8000


def _mlp_block(x_ref, rbf_ref, wrbf_ref, wup_ref, w1_ref, b1_ref, w2_ref,
               b2_ref, wout_ref, o_ref):
    wa = jnp.dot(wup_ref[...], w1_ref[...],
                 preferred_element_type=jnp.float32)
    h = jnp.dot(rbf_ref[...], wrbf_ref[...],
                preferred_element_type=jnp.float32) * x_ref[...]
    o = jax.nn.silu(jnp.dot(h, wa,
                            preferred_element_type=jnp.float32) + b1_ref[...])
    o = jax.nn.silu(jnp.dot(o, w2_ref[...],
                            preferred_element_type=jnp.float32) + b2_ref[...])
    o_ref[...] = jnp.dot(o, wout_ref[...], preferred_element_type=jnp.float32)


def _blockdiag(w):
    z = jnp.zeros_like(w)
    return jnp.concatenate(
        [jnp.concatenate([w, z], axis=1), jnp.concatenate([z, w], axis=1)],
        axis=0)


def kernel(x, rbf, i, num_nodes, W_rbf, W_up, W1, b1, W2, b2, W_out):
    del i, num_nodes
    E, H = x.shape
    R = rbf.shape[1]
    D = W_up.shape[1]
    P = E // 2

    x2 = x.reshape(P, 2 * H)
    rbf2 = rbf.reshape(P, 2 * R)
    Wrbf2 = _blockdiag(W_rbf)
    Wup2 = _blockdiag(W_up)
    W12 = _blockdiag(W1)
    W22 = _blockdiag(W2)
    Wout2 = _blockdiag(W_out)
    b12 = jnp.concatenate([b1, b1]).reshape(1, 2 * D)
    b22 = jnp.concatenate([b2, b2]).reshape(1, 2 * D)

    grid = (P // _BLOCK,)
    row_spec = lambda shape: pl.BlockSpec(shape, lambda m: (m, 0))
    rep_spec = lambda shape: pl.BlockSpec(shape, lambda m: (0, 0))

    out2 = pl.pallas_call(
        _mlp_block,
        grid=grid,
        in_specs=[
            row_spec((_BLOCK, 2 * H)),
            row_spec((_BLOCK, 2 * R)),
            rep_spec((2 * R, 2 * H)),
            rep_spec((2 * H, 2 * D)),
            rep_spec((2 * D, 2 * D)),
            rep_spec((1, 2 * D)),
            rep_spec((2 * D, 2 * D)),
            rep_spec((1, 2 * D)),
            rep_spec((2 * D, 2)),
        ],
        out_specs=row_spec((_BLOCK, 2)),
        out_shape=jax.ShapeDtypeStruct((P, 2), jnp.float32),
    )(x2, rbf2, Wrbf2, Wup2, W12, b12, W22, b22, Wout2)
    return out2.reshape(E, 1)


# speedup vs baseline: 1.6976x; 1.6976x over previous
import jax
import jax.numpy as jnp
from jax.experimental import pallas as pl
from jax.experimental.pallas import tpu as pltpu

_BLOCK = 8000


def _mlp_block(x_ref, rbf_ref, wrbf_ref, wup_ref, w1_ref, b1_ref, w2_ref,
               b2_ref, wout_ref, o_ref):
    wa = jnp.dot(wup_ref[...], w1_ref[...],
                 preferred_element_type=jnp.float32)
    h = jnp.dot(rbf_ref[...], wrbf_ref[...],
                preferred_element_type=jnp.float32) * x_ref[...]
    o = jax.nn.silu(jnp.dot(h, wa,
                            preferred_element_type=jnp.float32) + b1_ref[...])
    o = jax.nn.silu(jnp.dot(o, w2_ref[...],
                            preferred_element_type=jnp.float32) + b2_ref[...])
    o_ref[...] = jnp.dot(o, wout_ref[...], preferred_element_type=jnp.float32)


def kernel(x, rbf, i, num_nodes, W_rbf, W_up, W1, b1, W2, b2, W_out):
    del i, num_nodes
    E, H = x.shape
    R = rbf.shape[1]
    D = W_up.shape[1]
    b1 = b1.reshape(1, D)
    b2 = b2.reshape(1, D)

    grid = (E // _BLOCK,)
    row_spec = lambda shape: pl.BlockSpec(shape, lambda m: (m, 0))
    rep_spec = lambda shape: pl.BlockSpec(shape, lambda m: (0, 0))

    return pl.pallas_call(
        _mlp_block,
        grid=grid,
        in_specs=[
            row_spec((_BLOCK, H)),
            row_spec((_BLOCK, R)),
            rep_spec((R, H)),
            rep_spec((H, D)),
            rep_spec((D, D)),
            rep_spec((1, D)),
            rep_spec((D, D)),
            rep_spec((1, D)),
            rep_spec((D, 1)),
        ],
        out_specs=row_spec((_BLOCK, 1)),
        out_shape=jax.ShapeDtypeStruct((E, 1), jnp.float32),
        compiler_params=pltpu.CompilerParams(
            dimension_semantics=("parallel",)),
    )(x, rbf, W_rbf, W_up, W1, b1, W2, b2, W_out)
